# baseline (device time: 33346 ns/iter reference)
import os

import numpy as np
import jax
import jax.numpy as jnp
from jax import lax
from jax.experimental import pallas as pl
from jax.experimental.pallas import tpu as pltpu

_SKIP_COMM = os.environ.get("SKIP_COMM") == "1"

N_DEV = 16
B, SQ, D = 2, 256, 768
CH = SQ // N_DEV
HQ_LOCAL, DH = 4, 64
HD = HQ_LOCAL * DH


def _rope_tables():
    inv = 1.0 / (10000.0 ** (np.arange(0, DH, 2) / DH))
    pos = np.arange(SQ)[:, None] * inv[None, :]
    cosv = np.cos(pos).astype(np.float32)
    sinv = np.sin(pos).astype(np.float32)
    cos_d = np.tile(np.concatenate([cosv, cosv], axis=1), (1, HQ_LOCAL))
    sin_d = np.tile(np.concatenate([sinv, sinv], axis=1), (1, HQ_LOCAL))
    return jnp.asarray(cos_d), jnp.asarray(sin_d)


def _deinterleave_cols(w):
    return w.reshape(D, HQ_LOCAL, DH // 2, 2).transpose(0, 1, 3, 2).reshape(D, HD)


def kernel(x, Wq, Wk, Wv, Wo):
    cos_d, sin_d = _rope_tables()
    xb = x.astype(jnp.bfloat16)
    wq = _deinterleave_cols(Wq).astype(jnp.bfloat16)
    wk = _deinterleave_cols(Wk).astype(jnp.bfloat16)
    wv = Wv.astype(jnp.bfloat16)
    wo = Wo.astype(jnp.bfloat16)

    def body(x_ref, wq_ref, wk_ref, wv_ref, wo_ref, cos_ref, sin_ref,
             out_ref, work_ref, land_ref,
             rs_send, rs_recv, ag_send, ag_recv):
        my = lax.axis_index("i")

        barrier_sem = pltpu.get_barrier_semaphore()
        for j in range(N_DEV):
            @pl.when(j != my)
            def _():
                pl.semaphore_signal(
                    barrier_sem, inc=1,
                    device_id=(jnp.int32(j),),
                    device_id_type=pl.DeviceIdType.MESH,
                )
        pl.semaphore_wait(barrier_sem, N_DEV - 1)

        cos = cos_ref[...]
        sin = sin_ref[...]

        def qkv(b):
            x_b = x_ref[b, :, :]
            q = jnp.dot(x_b, wq_ref[...], preferred_element_type=jnp.float32)
            k = jnp.dot(x_b, wk_ref[...], preferred_element_type=jnp.float32)
            v = jnp.dot(x_b, wv_ref[...],
                        preferred_element_type=jnp.float32).astype(jnp.bfloat16)

            def rope(t):
                parts = []
                for h in range(HQ_LOCAL):
                    lo, mid, hi = h * DH, h * DH + DH // 2, (h + 1) * DH
                    parts.append(-t[:, mid:hi])
                    parts.append(t[:, lo:mid])
                return t * cos + jnp.concatenate(parts, axis=1) * sin

            return rope(q).astype(jnp.bfloat16), rope(k).astype(jnp.bfloat16), v

        N_WAVE = 2
        WSQ = SQ // N_WAVE
        WCH = N_DEV // N_WAVE

        def attn_wave(b, wv_, qkv_b):
            q, k, v = qkv_b
            rows = slice(wv_ * WSQ, (wv_ + 1) * WSQ)
            ctxs = []
            for h in range(HQ_LOCAL):
                sl = slice(h * DH, (h + 1) * DH)
                s_ = lax.dot_general(
                    q[rows, sl], k[:, sl], (((1,), (1,)), ((), ())),
                    preferred_element_type=jnp.float32) * 0.125
                e = jnp.exp(s_)
                w = (e / jnp.sum(e, axis=1, keepdims=True)).astype(jnp.bfloat16)
                ctxs.append(jnp.dot(w, v[:, sl],
                                    preferred_element_type=jnp.float32)
                            .astype(jnp.bfloat16))
            ctx = jnp.concatenate(ctxs, axis=1)
            part = jnp.dot(ctx, wo_ref[...], preferred_element_type=jnp.float32)
            work_ref[b, rows, :] = part.astype(jnp.bfloat16)
            for d in range(wv_ * WCH, (wv_ + 1) * WCH):
                @pl.when(d != my)
                def _():
                    pltpu.make_async_remote_copy(
                        src_ref=work_ref.at[b, pl.ds(d * CH, CH)],
                        dst_ref=land_ref.at[b, my],
                        send_sem=rs_send.at[b, d],
                        recv_sem=rs_recv.at[b, my],
                        device_id=(jnp.int32(d),),
                        device_id_type=pl.DeviceIdType.MESH,
                    ).start()

        def rs_reduce_and_ag_send(b):
            for j in range(N_DEV):
                @pl.when(j != my)
                def _():
                    pltpu.make_async_remote_copy(
                        src_ref=work_ref.at[b, pl.ds(j * CH, CH)],
                        dst_ref=land_ref.at[b, j],
                        send_sem=rs_send.at[b, j],
                        recv_sem=rs_recv.at[b, j],
                        device_id=(jnp.int32(j),),
                        device_id_type=pl.DeviceIdType.MESH,
                    ).wait_recv()
            red = work_ref[b, pl.ds(my * CH, CH), :].astype(jnp.float32)
            for j in range(N_DEV):
                red = red + jnp.where(
                    j == my, jnp.float32(0),
                    land_ref[b, j, :, :].astype(jnp.float32))
            out_ref[b, pl.ds(my * CH, CH), :] = red.astype(jnp.bfloat16)
            for d in range(N_DEV):
                @pl.when(d != my)
                def _():
                    pltpu.make_async_remote_copy(
                        src_ref=out_ref.at[b, pl.ds(my * CH, CH)],
                        dst_ref=out_ref.at[b, pl.ds(my * CH, CH)],
                        send_sem=ag_send.at[b, d],
                        recv_sem=ag_recv.at[b, my],
                        device_id=(jnp.int32(d),),
                        device_id_type=pl.DeviceIdType.MESH,
                    ).start()

        def ag_wait(b):
            for j in range(N_DEV):
                @pl.when(j != my)
                def _():
                    pltpu.make_async_remote_copy(
                        src_ref=out_ref.at[b, pl.ds(j * CH, CH)],
                        dst_ref=out_ref.at[b, pl.ds(j * CH, CH)],
                        send_sem=ag_send.at[b, j],
                        recv_sem=ag_recv.at[b, j],
                        device_id=(jnp.int32(j),),
                        device_id_type=pl.DeviceIdType.MESH,
                    ).wait_recv()

        if _SKIP_COMM:
            for b in range(B):
                qkv_b = qkv(b)
                for wv_ in range(N_WAVE):
                    attn_wave(b, wv_, qkv_b)
                out_ref[b, :, :] = work_ref[b, :, :]
            return

        qkv0 = qkv(0)
        attn_wave(0, 0, qkv0)
        attn_wave(0, 1, qkv0)
        qkv1 = qkv(1)
        attn_wave(1, 0, qkv1)
        attn_wave(1, 1, qkv1)
        rs_reduce_and_ag_send(0)
        rs_reduce_and_ag_send(1)
        ag_wait(0)
        ag_wait(1)

        for b in range(B):
            for d in range(N_DEV):
                @pl.when(d != my)
                def _():
                    pltpu.make_async_remote_copy(
                        src_ref=work_ref.at[b, pl.ds(d * CH, CH)],
                        dst_ref=land_ref.at[b, my],
                        send_sem=rs_send.at[b, d],
                        recv_sem=rs_recv.at[b, my],
                        device_id=(jnp.int32(d),),
                        device_id_type=pl.DeviceIdType.MESH,
                    ).wait_send()
                    pltpu.make_async_remote_copy(
                        src_ref=out_ref.at[b, pl.ds(my * CH, CH)],
                        dst_ref=out_ref.at[b, pl.ds(my * CH, CH)],
                        send_sem=ag_send.at[b, d],
                        recv_sem=ag_recv.at[b, my],
                        device_id=(jnp.int32(d),),
                        device_id_type=pl.DeviceIdType.MESH,
                    ).wait_send()

    return pl.pallas_call(
        body,
        out_shape=jax.ShapeDtypeStruct((B, SQ, D), jnp.bfloat16),
        in_specs=[pl.BlockSpec(memory_space=pltpu.VMEM)] * 7,
        out_specs=pl.BlockSpec(memory_space=pltpu.VMEM),
        scratch_shapes=[
            pltpu.VMEM((B, SQ, D), jnp.bfloat16),
            pltpu.VMEM((B, N_DEV, CH, D), jnp.bfloat16),
            pltpu.SemaphoreType.DMA((B, N_DEV)),
            pltpu.SemaphoreType.DMA((B, N_DEV)),
            pltpu.SemaphoreType.DMA((B, N_DEV)),
            pltpu.SemaphoreType.DMA((B, N_DEV)),
        ],
        compiler_params=pltpu.CompilerParams(collective_id=0),
    )(xb, wq, wk, wv, wo, cos_d, sin_d)


# device time: 31500 ns/iter; 1.0586x vs baseline; 1.0586x over previous
import os

import numpy as np
import jax
import jax.numpy as jnp
from jax import lax
from jax.experimental import pallas as pl
from jax.experimental.pallas import tpu as pltpu

_SKIP_COMM = os.environ.get("SKIP_COMM") == "1"
_PROF = os.environ.get("PROFILE_SCOPES") == "1"


class _NullCtx:
    def __enter__(self):
        return None

    def __exit__(self, *a):
        return False


def _scope(name):
    return jax.named_scope(name) if _PROF else _NullCtx()

N_DEV = 16
B, SQ, D = 2, 256, 768
CH = SQ // N_DEV
HQ_LOCAL, DH = 4, 64
HD = HQ_LOCAL * DH


def _rope_tables():
    inv = 1.0 / (10000.0 ** (np.arange(0, DH, 2) / DH))
    pos = np.arange(SQ)[:, None] * inv[None, :]
    cosv = np.cos(pos).astype(np.float32)
    sinv = np.sin(pos).astype(np.float32)
    cos_d = np.tile(np.concatenate([cosv, cosv], axis=1), (1, HQ_LOCAL))
    sin_d = np.tile(np.concatenate([sinv, sinv], axis=1), (1, HQ_LOCAL))
    return jnp.asarray(cos_d), jnp.asarray(sin_d)


def _deinterleave_cols(w):
    return w.reshape(D, HQ_LOCAL, DH // 2, 2).transpose(0, 1, 3, 2).reshape(D, HD)


def kernel(x, Wq, Wk, Wv, Wo):
    cos_d, sin_d = _rope_tables()
    xb = x.astype(jnp.bfloat16)
    wq = _deinterleave_cols(Wq).astype(jnp.bfloat16)
    wk = _deinterleave_cols(Wk).astype(jnp.bfloat16)
    wv = Wv.astype(jnp.bfloat16)
    wo = Wo.astype(jnp.bfloat16)

    def body(x_ref, wq_ref, wk_ref, wv_ref, wo_ref, cos_ref, sin_ref,
             out_ref, work_ref, land_ref,
             rs_send, rs_recv, ag_send, ag_recv):
        my = lax.axis_index("i")

        barrier_sem = pltpu.get_barrier_semaphore()
        with _scope("pA_barrier_signal"):
            for j in range(N_DEV):
                @pl.when(j != my)
                def _():
                    pl.semaphore_signal(
                        barrier_sem, inc=1,
                        device_id=(jnp.int32(j),),
                        device_id_type=pl.DeviceIdType.MESH,
                    )

        cos = cos_ref[...]
        sin = sin_ref[...]

        def qkv(b):
            x_b = x_ref[b, :, :]
            q = jnp.dot(x_b, wq_ref[...], preferred_element_type=jnp.float32)
            k = jnp.dot(x_b, wk_ref[...], preferred_element_type=jnp.float32)
            v = jnp.dot(x_b, wv_ref[...],
                        preferred_element_type=jnp.float32).astype(jnp.bfloat16)

            def rope(t):
                parts = []
                for h in range(HQ_LOCAL):
                    lo, mid, hi = h * DH, h * DH + DH // 2, (h + 1) * DH
                    parts.append(-t[:, mid:hi])
                    parts.append(t[:, lo:mid])
                return t * cos + jnp.concatenate(parts, axis=1) * sin

            return rope(q).astype(jnp.bfloat16), rope(k).astype(jnp.bfloat16), v

        N_WAVE = 2
        WSQ = SQ // N_WAVE
        WCH = N_DEV // N_WAVE

        def attn_wave(b, wv_, qkv_b, pre_send=None):
            q, k, v = qkv_b
            rows = slice(wv_ * WSQ, (wv_ + 1) * WSQ)
            ctxs = []
            for h in range(HQ_LOCAL):
                sl = slice(h * DH, (h + 1) * DH)
                s_ = lax.dot_general(
                    q[rows, sl], k[:, sl], (((1,), (1,)), ((), ())),
                    preferred_element_type=jnp.float32) * 0.125
                e = jnp.exp(s_)
                w = (e / jnp.sum(e, axis=1, keepdims=True)).astype(jnp.bfloat16)
                ctxs.append(jnp.dot(w, v[:, sl],
                                    preferred_element_type=jnp.float32)
                            .astype(jnp.bfloat16))
            ctx = jnp.concatenate(ctxs, axis=1)
            part = jnp.dot(ctx, wo_ref[...], preferred_element_type=jnp.float32)
            work_ref[b, rows, :] = part.astype(jnp.bfloat16)
            if pre_send is not None:
                pre_send()
            for d in range(wv_ * WCH, (wv_ + 1) * WCH):
                @pl.when(d != my)
                def _():
                    pltpu.make_async_remote_copy(
                        src_ref=work_ref.at[b, pl.ds(d * CH, CH)],
                        dst_ref=land_ref.at[b, my],
                        send_sem=rs_send.at[b, d],
                        recv_sem=rs_recv.at[b, my],
                        device_id=(jnp.int32(d),),
                        device_id_type=pl.DeviceIdType.MESH,
                    ).start()

        def rs_reduce_and_ag_send(b):
            for j in range(N_DEV):
                @pl.when(j != my)
                def _():
                    pltpu.make_async_remote_copy(
                        src_ref=work_ref.at[b, pl.ds(j * CH, CH)],
                        dst_ref=land_ref.at[b, j],
                        send_sem=rs_send.at[b, j],
                        recv_sem=rs_recv.at[b, j],
                        device_id=(jnp.int32(j),),
                        device_id_type=pl.DeviceIdType.MESH,
                    ).wait_recv()
            red = work_ref[b, pl.ds(my * CH, CH), :].astype(jnp.float32)
            for j in range(N_DEV):
                red = red + jnp.where(
                    j == my, jnp.float32(0),
                    land_ref[b, j, :, :].astype(jnp.float32))
            out_ref[b, pl.ds(my * CH, CH), :] = red.astype(jnp.bfloat16)
            for d in range(N_DEV):
                @pl.when(d != my)
                def _():
                    pltpu.make_async_remote_copy(
                        src_ref=out_ref.at[b, pl.ds(my * CH, CH)],
                        dst_ref=out_ref.at[b, pl.ds(my * CH, CH)],
                        send_sem=ag_send.at[b, d],
                        recv_sem=ag_recv.at[b, my],
                        device_id=(jnp.int32(d),),
                        device_id_type=pl.DeviceIdType.MESH,
                    ).start()

        def ag_wait(b):
            for j in range(N_DEV):
                @pl.when(j != my)
                def _():
                    pltpu.make_async_remote_copy(
                        src_ref=out_ref.at[b, pl.ds(j * CH, CH)],
                        dst_ref=out_ref.at[b, pl.ds(j * CH, CH)],
                        send_sem=ag_send.at[b, j],
                        recv_sem=ag_recv.at[b, j],
                        device_id=(jnp.int32(j),),
                        device_id_type=pl.DeviceIdType.MESH,
                    ).wait_recv()

        if _SKIP_COMM:
            pl.semaphore_wait(barrier_sem, N_DEV - 1)
            for b in range(B):
                qkv_b = qkv(b)
                for wv_ in range(N_WAVE):
                    attn_wave(b, wv_, qkv_b)
                out_ref[b, :, :] = work_ref[b, :, :]
            return

        def barrier_wait():
            with _scope("pC_barrier_wait"):
                pl.semaphore_wait(barrier_sem, N_DEV - 1)

        with _scope("p0_qkv0"):
            qkv0 = qkv(0)
        with _scope("p1_wave00"):
            attn_wave(0, 0, qkv0, pre_send=barrier_wait)
        with _scope("p2_wave01"):
            attn_wave(0, 1, qkv0)
        with _scope("p3_qkv1"):
            qkv1 = qkv(1)
        with _scope("p4_wave10"):
            attn_wave(1, 0, qkv1)
        with _scope("p5_wave11"):
            attn_wave(1, 1, qkv1)
        with _scope("p6_rs_red_ag0"):
            rs_reduce_and_ag_send(0)
        with _scope("p7_rs_red_ag1"):
            rs_reduce_and_ag_send(1)
        with _scope("p8_agw0"):
            ag_wait(0)
        with _scope("p9_agw1"):
            ag_wait(1)

        drain_scope = _scope("pB_drain")
        drain_scope.__enter__()
        for b in range(B):
            for d in range(N_DEV):
                @pl.when(d != my)
                def _():
                    pltpu.make_async_remote_copy(
                        src_ref=work_ref.at[b, pl.ds(d * CH, CH)],
                        dst_ref=land_ref.at[b, my],
                        send_sem=rs_send.at[b, d],
                        recv_sem=rs_recv.at[b, my],
                        device_id=(jnp.int32(d),),
                        device_id_type=pl.DeviceIdType.MESH,
                    ).wait_send()
                    pltpu.make_async_remote_copy(
                        src_ref=out_ref.at[b, pl.ds(my * CH, CH)],
                        dst_ref=out_ref.at[b, pl.ds(my * CH, CH)],
                        send_sem=ag_send.at[b, d],
                        recv_sem=ag_recv.at[b, my],
                        device_id=(jnp.int32(d),),
                        device_id_type=pl.DeviceIdType.MESH,
                    ).wait_send()
        drain_scope.__exit__(None, None, None)

    return pl.pallas_call(
        body,
        out_shape=jax.ShapeDtypeStruct((B, SQ, D), jnp.bfloat16),
        in_specs=[pl.BlockSpec(memory_space=pltpu.VMEM)] * 7,
        out_specs=pl.BlockSpec(memory_space=pltpu.VMEM),
        scratch_shapes=[
            pltpu.VMEM((B, SQ, D), jnp.bfloat16),
            pltpu.VMEM((B, N_DEV, CH, D), jnp.bfloat16),
            pltpu.SemaphoreType.DMA((B, N_DEV)),
            pltpu.SemaphoreType.DMA((B, N_DEV)),
            pltpu.SemaphoreType.DMA((B, N_DEV)),
            pltpu.SemaphoreType.DMA((B, N_DEV)),
        ],
        compiler_params=pltpu.CompilerParams(collective_id=0),
    )(xb, wq, wk, wv, wo, cos_d, sin_d)
